# Initial kernel scaffold; baseline (speedup 1.0000x reference)
import jax
import jax.numpy as jnp
from jax.experimental import pallas as pl


def kernel(x_type, x_sub, x_bow, x_hot, x_topo, edge_index, edge_type, batch, g_feats, emb_type, emb_sub, comp1, basis1, root1, bias1, comp2, basis2, root2, bias2, comp3, basis3, root3, bias3, bn_gamma, bn_beta, gate_w1, gate_b1, gate_w2, gate_b2, head_pt_w, head_pt_b, head_ci_w, head_ci_b):
    raise NotImplementedError("write your pallas kernel here")


# speedup vs baseline: 8.8064x; 8.8064x over previous
import functools

import jax
import jax.numpy as jnp
from jax import lax
from jax.experimental import pallas as pl
from jax.experimental.pallas import tpu as pltpu
from jax.experimental.pallas import tpu_sc as plsc

_f32 = jnp.float32
_i32 = jnp.int32

N = 50000
E = 800000
R = 8
HID = 64
G = 16
NINE = (R + 1) * HID

HALF = 25000
PT = 1568
HPAD = 16 * PT
TRASH = HALF
CT = HPAD * 8
CZT = CT // 16
CTRASH = HALF * 8

EPAD = 802816
EROWS = EPAD // 128
RPT = EROWS // 16
NCHUNK = RPT // 4
RPT32 = EROWS // 32
NCHUNK32 = RPT32 // 4

BN_ = 2000
NB = N // BN_


def _sc_mesh():
  return plsc.VectorSubcoreMesh(core_axis_name="c", subcore_axis_name="s",
                                num_cores=2, num_subcores=16)


_SC_PARAMS = pltpu.CompilerParams(use_tc_tiling_on_sc=False)


def _sc_counts_body(cidx_ref, cnt_out_ref, cnt_sh, idxv, onesv, zb):
  c = lax.axis_index("c")
  s = lax.axis_index("s")
  zero16 = jnp.zeros((16,), _f32)

  def zl(i, carry):
    zb[pl.ds(i * 16, 16)] = zero16
    return carry

  lax.fori_loop(0, CZT // 16, zl, 0)
  for t in range(8):
    onesv[pl.ds(t * 16, 16)] = jnp.ones((16,), _f32)
  pltpu.sync_copy(zb, cnt_sh.at[pl.ds(s * CZT, CZT)])
  plsc.subcore_barrier()

  def chunk(i, carry):
    r0 = s * RPT + i * 4
    pltpu.sync_copy(cidx_ref.at[c, pl.ds(r0, 4)], idxv)
    for j in range(4):
      pltpu.sync_copy(onesv, cnt_sh.at[idxv.at[j]], add=True)
    return carry

  lax.fori_loop(0, NCHUNK, chunk, 0)
  plsc.subcore_barrier()
  pltpu.sync_copy(cnt_sh.at[pl.ds(s * CZT, CZT)],
                  cnt_out_ref.at[pl.ds(c * CT + s * CZT, CZT)])


def _sc_counts(cidx2):
  return pl.kernel(
      _sc_counts_body,
      out_type=jax.ShapeDtypeStruct((2 * CT,), _f32),
      mesh=_sc_mesh(),
      scratch_types=[
          pltpu.MemorySpace.VMEM_SHARED((CT,), _f32),
          pltpu.MemorySpace.VMEM((4, 128), _i32),
          pltpu.MemorySpace.VMEM((128,), _f32),
          pltpu.MemorySpace.VMEM((CZT,), _f32),
      ],
      compiler_params=_SC_PARAMS,
      name="sc_counts",
  )(cidx2)


def _sc_scale_body(sgidx_ref, cnt_ref, scale_out_ref, idxv, cv, sv, sem):
  c = lax.axis_index("c")
  s = lax.axis_index("s")
  w = s * 2 + c

  def chunk(i, carry):
    r0 = w * RPT32 + i * 4
    pltpu.sync_copy(sgidx_ref.at[pl.ds(r0, 4)], idxv)
    cps = [pltpu.async_copy(cnt_ref.at[idxv.at[j]], cv.at[j], sem)
           for j in range(4)]
    for cp in cps:
      cp.wait()
    for j in range(4):
      for t in range(8):
        sl = pl.ds(t * 16, 16)
        sv[j, sl] = 1.0 / jnp.maximum(cv[j, sl], 1.0)
    pltpu.sync_copy(sv, scale_out_ref.at[pl.ds(r0, 4)])
    return carry

  lax.fori_loop(0, NCHUNK32, chunk, 0)


def _sc_scale(sgidx2, cnt):
  return pl.kernel(
      _sc_scale_body,
      out_type=jax.ShapeDtypeStruct((EROWS, 128), _f32),
      mesh=_sc_mesh(),
      scratch_types=[
          pltpu.MemorySpace.VMEM((4, 128), _i32),
          pltpu.MemorySpace.VMEM((4, 128), _f32),
          pltpu.MemorySpace.VMEM((4, 128), _f32),
          pltpu.SemaphoreType.DMA,
      ],
      compiler_params=_SC_PARAMS,
      name="sc_scale",
  )(sgidx2, cnt)


def _sc_msg_body(xw_ref, gidx_ref, scl_ref, ldst_ref, acc_out_ref,
                 acc_sh, idxg, idxs, sclv, rows, sem):
  c = lax.axis_index("c")
  s = lax.axis_index("s")
  base = s * PT
  zero16 = jnp.zeros((16,), _f32)

  def zl(i, carry):
    for t in range(4):
      rows[i, pl.ds(t * 16, 16)] = zero16
    return carry

  lax.fori_loop(0, 256, zl, 0)
  for k in range(6):
    pltpu.sync_copy(rows, acc_sh.at[pl.ds(base + k * 256, 256)])
  pltpu.sync_copy(rows.at[pl.ds(0, PT - 1536)],
                  acc_sh.at[pl.ds(base + 1536, PT - 1536)])
  plsc.subcore_barrier()

  def chunk(i, carry):
    r0 = s * RPT + i * 2
    pltpu.sync_copy(gidx_ref.at[pl.ds(r0, 2)], idxg)
    pltpu.sync_copy(scl_ref.at[pl.ds(r0 * 128, 256)], sclv)
    pltpu.sync_copy(ldst_ref.at[c, pl.ds(r0, 2)], idxs)
    cps = [pltpu.async_copy(xw_ref.at[idxg.at[j]],
                            rows.at[pl.ds(j * 128, 128)], sem)
           for j in range(2)]
    for cp in cps:
      cp.wait()

    def grp(gi, gcarry):
      svec = sclv[pl.ds(gi * 16, 16)]
      for j in range(16):
        sc = jnp.take_along_axis(svec, jnp.full((16,), j, _i32), axis=0)
        e = gi * 16 + j
        for t in range(4):
          sl = pl.ds(t * 16, 16)
          rows[e, sl] = rows[e, sl] * sc
      return gcarry

    lax.fori_loop(0, 16, grp, 0)
    for j in range(2):
      pltpu.sync_copy(rows.at[pl.ds(j * 128, 128)],
                      acc_sh.at[idxs.at[j]], add=True)
    return carry

  lax.fori_loop(0, NCHUNK * 2, chunk, 0)
  plsc.subcore_barrier()
  pltpu.sync_copy(acc_sh.at[pl.ds(base, PT)],
                  acc_out_ref.at[pl.ds(c * HPAD + base, PT)])


def _sc_msg(xw_flat, gidx2, scale1, ldst2):
  return pl.kernel(
      _sc_msg_body,
      out_type=jax.ShapeDtypeStruct((2 * HPAD, HID), _f32),
      mesh=_sc_mesh(),
      scratch_types=[
          pltpu.MemorySpace.VMEM_SHARED((HPAD, HID), _f32),
          pltpu.MemorySpace.VMEM((2, 128), _i32),
          pltpu.MemorySpace.VMEM((2, 128), _i32),
          pltpu.MemorySpace.VMEM((256,), _f32),
          pltpu.MemorySpace.VMEM((256, HID), _f32),
          pltpu.SemaphoreType.DMA,
      ],
      compiler_params=_SC_PARAMS,
      name="sc_msg",
  )(xw_flat, gidx2, scale1, ldst2)


def _tc_layer1(xt3, xs3, bow, hot, topo, wa, wb, wcb, wch, wct):
  def body(xt_ref, xs_ref, bow_ref, hot_ref, topo_ref, wa_ref, wb_ref,
           wcb_ref, wch_ref, wct_ref, out0_ref, xw_ref):
    xt = xt_ref[0, 0, :]
    xs = xs_ref[0, 0, :]
    oht = (xt[:, None] == lax.broadcasted_iota(_i32, (1, 16), 1)).astype(_f32)
    ohs = (xs[:, None] == lax.broadcasted_iota(_i32, (1, 33), 1)).astype(_f32)
    res = (jnp.dot(oht, wa_ref[...], preferred_element_type=_f32)
           + jnp.dot(ohs, wb_ref[...], preferred_element_type=_f32)
           + jnp.dot(bow_ref[...], wcb_ref[...], preferred_element_type=_f32)
           + jnp.dot(hot_ref[...], wch_ref[...], preferred_element_type=_f32)
           + jnp.dot(topo_ref[...], wct_ref[...], preferred_element_type=_f32))
    out0_ref[...] = res[:, :HID]
    for r in range(R):
      xw_ref[r] = res[:, HID * (r + 1):HID * (r + 2)]

  return pl.pallas_call(
      body,
      grid=(NB,),
      in_specs=[
          pl.BlockSpec((1, 1, BN_), lambda i: (i, 0, 0)),
          pl.BlockSpec((1, 1, BN_), lambda i: (i, 0, 0)),
          pl.BlockSpec((BN_, 64), lambda i: (i, 0)),
          pl.BlockSpec((BN_, 10), lambda i: (i, 0)),
          pl.BlockSpec((BN_, 3), lambda i: (i, 0)),
          pl.BlockSpec((16, NINE), lambda i: (0, 0)),
          pl.BlockSpec((33, NINE), lambda i: (0, 0)),
          pl.BlockSpec((64, NINE), lambda i: (0, 0)),
          pl.BlockSpec((10, NINE), lambda i: (0, 0)),
          pl.BlockSpec((3, NINE), lambda i: (0, 0)),
      ],
      out_specs=[
          pl.BlockSpec((BN_, HID), lambda i: (i, 0)),
          pl.BlockSpec((R, BN_, HID), lambda i: (0, i, 0)),
      ],
      out_shape=[
          jax.ShapeDtypeStruct((N, HID), _f32),
          jax.ShapeDtypeStruct((R, N, HID), _f32),
      ],
      name="tc_layer1",
  )(xt3, xs3, bow, hot, topo, wa, wb, wcb, wch, wct)


def _norm_block(o_ref, a_ref, st_ref, g_ref, b_ref):
  h = o_ref[...] + a_ref[...]
  mu = st_ref[0:1, :] * (1.0 / N)
  msq = st_ref[1:2, :] * (1.0 / N)
  var = msq - mu * mu
  xn = g_ref[...] * (h - mu) * lax.rsqrt(var + 1e-5) + b_ref[...]
  return jnp.maximum(xn, 0.0)


def _tc_layer(out0p, accp, stats, gamma2, beta2, wr):
  def body(o_ref, a_ref, st_ref, g_ref, b_ref, wr_ref, out0_ref, xw_ref):
    x = _norm_block(o_ref, a_ref, st_ref, g_ref, b_ref)
    res = jnp.dot(x, wr_ref[...], preferred_element_type=_f32)
    out0_ref[...] = res[:, :HID]
    for r in range(R):
      xw_ref[r] = res[:, HID * (r + 1):HID * (r + 2)]

  return pl.pallas_call(
      body,
      grid=(NB,),
      in_specs=[
          pl.BlockSpec((BN_, HID), lambda i: (i, 0)),
          pl.BlockSpec((BN_, HID), lambda i: (i, 0)),
          pl.BlockSpec((8, HID), lambda i: (0, 0)),
          pl.BlockSpec((1, HID), lambda i: (0, 0)),
          pl.BlockSpec((1, HID), lambda i: (0, 0)),
          pl.BlockSpec((HID, NINE), lambda i: (0, 0)),
      ],
      out_specs=[
          pl.BlockSpec((BN_, HID), lambda i: (i, 0)),
          pl.BlockSpec((R, BN_, HID), lambda i: (0, i, 0)),
      ],
      out_shape=[
          jax.ShapeDtypeStruct((N, HID), _f32),
          jax.ShapeDtypeStruct((R, N, HID), _f32),
      ],
      name="tc_layer",
  )(out0p, accp, stats, gamma2, beta2, wr)


def _tc_stats(out0p, accp):
  def body(o_ref, a_ref, st_ref):
    i = pl.program_id(0)
    h = o_ref[...] + a_ref[...]

    @pl.when(i == 0)
    def _():
      st_ref[...] = jnp.zeros((8, HID), _f32)

    st_ref[0:1, :] = st_ref[0:1, :] + jnp.sum(h, axis=0, keepdims=True)
    st_ref[1:2, :] = st_ref[1:2, :] + jnp.sum(h * h, axis=0, keepdims=True)

  return pl.pallas_call(
      body,
      grid=(NB,),
      in_specs=[
          pl.BlockSpec((BN_, HID), lambda i: (i, 0)),
          pl.BlockSpec((BN_, HID), lambda i: (i, 0)),
      ],
      out_specs=pl.BlockSpec((8, HID), lambda i: (0, 0)),
      out_shape=jax.ShapeDtypeStruct((8, HID), _f32),
      name="tc_stats",
  )(out0p, accp)


def _tc_gate(out0p, accp, stats, gamma2, beta2, gw1, gb1, gw2p, gb2p, batch3):
  def body(o_ref, a_ref, st_ref, g_ref, b_ref, w1_ref, b1_ref, w2_ref,
           b2_ref, bt_ref, x3_ref, gate_ref, m_ref):
    i = pl.program_id(0)
    x = _norm_block(o_ref, a_ref, st_ref, g_ref, b_ref)
    x3_ref[...] = x
    g1 = jnp.maximum(
        jnp.dot(x, w1_ref[...], preferred_element_type=_f32) + b1_ref[...],
        0.0)
    gt = jnp.dot(g1, w2_ref[...], preferred_element_type=_f32) + b2_ref[...]
    gate_ref[...] = gt
    bt = bt_ref[0, 0, :]
    oh = bt[:, None] == lax.broadcasted_iota(_i32, (1, 128), 1)
    mp = jnp.max(jnp.where(oh, gt[:, 0:1], -1e30), axis=0, keepdims=True)

    @pl.when(i == 0)
    def _():
      m_ref[...] = jnp.full((8, 128), -1e30, _f32)

    m_ref[...] = jnp.maximum(m_ref[...], jnp.broadcast_to(mp, (8, 128)))

  return pl.pallas_call(
      body,
      grid=(NB,),
      in_specs=[
          pl.BlockSpec((BN_, HID), lambda i: (i, 0)),
          pl.BlockSpec((BN_, HID), lambda i: (i, 0)),
          pl.BlockSpec((8, HID), lambda i: (0, 0)),
          pl.BlockSpec((1, HID), lambda i: (0, 0)),
          pl.BlockSpec((1, HID), lambda i: (0, 0)),
          pl.BlockSpec((HID, 64), lambda i: (0, 0)),
          pl.BlockSpec((1, 64), lambda i: (0, 0)),
          pl.BlockSpec((64, 128), lambda i: (0, 0)),
          pl.BlockSpec((1, 128), lambda i: (0, 0)),
          pl.BlockSpec((1, 1, BN_), lambda i: (i, 0, 0)),
      ],
      out_specs=[
          pl.BlockSpec((BN_, HID), lambda i: (i, 0)),
          pl.BlockSpec((BN_, 128), lambda i: (i, 0)),
          pl.BlockSpec((8, 128), lambda i: (0, 0)),
      ],
      out_shape=[
          jax.ShapeDtypeStruct((N, HID), _f32),
          jax.ShapeDtypeStruct((N, 128), _f32),
          jax.ShapeDtypeStruct((8, 128), _f32),
      ],
      name="tc_gate",
  )(out0p, accp, stats, gamma2, beta2, gw1, gb1, gw2p, gb2p, batch3)


def _tc_pool(x3, gate, m, batch3, gfp, h1, h2, bv):
  def body(x_ref, gt_ref, m_ref, bt_ref, gf_ref, h1_ref, h2_ref, bv_ref,
           out_ref, zacc):
    i = pl.program_id(0)
    bt = bt_ref[0, 0, :]
    oh16 = (bt[:, None] == lax.broadcasted_iota(_i32, (1, G), 1)).astype(_f32)
    m16 = m_ref[0:1, 0:G]
    mb = jnp.sum(oh16 * m16, axis=1, keepdims=True)
    e = jnp.exp(gt_ref[:, 0:1] - mb)
    z = jnp.concatenate(
        [e * x_ref[...], jnp.broadcast_to(e, (BN_, HID))], axis=1)
    part = lax.dot_general(oh16, z, (((0,), (0,)), ((), ())),
                           preferred_element_type=_f32)

    @pl.when(i == 0)
    def _():
      zacc[...] = jnp.zeros((G, 128), _f32)

    zacc[...] = zacc[...] + part

    @pl.when(i == NB - 1)
    def _():
      u = zacc[...]
      sval = u[:, HID:HID + 1]
      gpool = u[:, 0:HID] / jnp.maximum(sval, 1e-30)
      lg = (jnp.dot(gpool, h1_ref[...], preferred_element_type=_f32)
            + jnp.dot(gf_ref[...], h2_ref[...], preferred_element_type=_f32)
            + bv_ref[...])
      out_ref[...] = lg

  return pl.pallas_call(
      body,
      grid=(NB,),
      in_specs=[
          pl.BlockSpec((BN_, HID), lambda i: (i, 0)),
          pl.BlockSpec((BN_, 128), lambda i: (i, 0)),
          pl.BlockSpec((8, 128), lambda i: (0, 0)),
          pl.BlockSpec((1, 1, BN_), lambda i: (i, 0, 0)),
          pl.BlockSpec((G, G), lambda i: (0, 0)),
          pl.BlockSpec((HID, 128), lambda i: (0, 0)),
          pl.BlockSpec((G, 128), lambda i: (0, 0)),
          pl.BlockSpec((1, 128), lambda i: (0, 0)),
      ],
      out_specs=pl.BlockSpec((G, 128), lambda i: (0, 0)),
      out_shape=jax.ShapeDtypeStruct((G, 128), _f32),
      scratch_shapes=[pltpu.MemorySpace.VMEM((G, 128), _f32)],
      name="tc_pool",
  )(x3, gate, m, batch3, gfp, h1, h2, bv)


def _wr_cat(comp, basis, root):
  w = jnp.einsum('rb,bio->rio', comp, basis)
  d = root.shape[0]
  return jnp.concatenate([root[None], w], axis=0).transpose(1, 0, 2).reshape(
      d, NINE)


def _pad_edges(a, fill):
  pad = EPAD - E
  return jnp.concatenate(
      [a, jnp.full((pad,), fill, a.dtype)]).reshape(EROWS, 128)


def kernel(x_type, x_sub, x_bow, x_hot, x_topo, edge_index, edge_type, batch,
           g_feats, emb_type, emb_sub, comp1, basis1, root1, bias1, comp2,
           basis2, root2, bias2, comp3, basis3, root3, bias3, bn_gamma,
           bn_beta, gate_w1, gate_b1, gate_w2, gate_b2, head_pt_w, head_pt_b,
           head_ci_w, head_ci_b):
  wr1 = _wr_cat(comp1, basis1, root1)
  wr2 = _wr_cat(comp2, basis2, root2)
  wr3 = _wr_cat(comp3, basis3, root3)
  wa = emb_type @ wr1[:16]
  wb = emb_sub @ wr1[16:24]
  wcb = wr1[24:88]
  wch = wr1[88:98]
  wct = wr1[98:101]
  gamma2 = bn_gamma.reshape(1, HID)
  beta2 = bn_beta.reshape(1, HID)
  gb1 = gate_b1.reshape(1, 64)
  gw2p = jnp.zeros((64, 128), _f32).at[:, 0].set(gate_w2[:, 0])
  gb2p = jnp.zeros((1, 128), _f32).at[0, 0].set(gate_b2[0])
  h1 = jnp.zeros((HID, 128), _f32)
  h1 = h1.at[:, 0].set(head_pt_w[:HID, 0]).at[:, 1].set(head_ci_w[:HID, 0])
  h2 = jnp.zeros((G, 128), _f32)
  h2 = h2.at[:12, 0].set(head_pt_w[HID:, 0]).at[:12, 1].set(head_ci_w[HID:, 0])
  gfp = jnp.zeros((G, G), _f32).at[:, :12].set(g_feats)
  bv = jnp.zeros((1, 128), _f32).at[0, 0].set(head_pt_b[0]).at[0, 1].set(
      head_ci_b[0])

  src = edge_index[0].astype(_i32)
  dst = edge_index[1].astype(_i32)
  et = edge_type.astype(_i32)
  in0 = dst < HALF
  gidx2 = _pad_edges(et * N + src, 0)
  ldst0 = jnp.where(in0, dst, TRASH)
  ldst1 = jnp.where(in0, TRASH, dst - HALF)
  ldst2 = jnp.stack([_pad_edges(ldst0, TRASH), _pad_edges(ldst1, TRASH)])
  cidx0 = jnp.where(in0, dst * 8 + et, CTRASH)
  cidx1 = jnp.where(in0, CTRASH, (dst - HALF) * 8 + et)
  cidx2 = jnp.stack([_pad_edges(cidx0, CTRASH), _pad_edges(cidx1, CTRASH)])
  cof = (~in0).astype(_i32)
  sgidx = cof * CT + (dst - cof * HALF) * 8 + et
  sgidx2 = _pad_edges(sgidx, 0)
  xt3 = x_type.astype(_i32).reshape(NB, 1, BN_)
  xs3 = x_sub.astype(_i32).reshape(NB, 1, BN_)
  batch3 = batch.astype(_i32).reshape(NB, 1, BN_)

  cnt = _sc_counts(cidx2)
  scale1 = _sc_scale(sgidx2, cnt).reshape(EPAD)

  out0, xw = _tc_layer1(xt3, xs3, x_bow, x_hot, x_topo,
                        wa, wb, wcb, wch, wct)
  wrs = (None, None, wr2, wr3)
  acc = None
  stats = None
  for l in (1, 2, 3):
    accp = _sc_msg(xw.reshape(R * N, HID), gidx2, scale1, ldst2)
    acc = jnp.concatenate([accp[:HALF], accp[HPAD:HPAD + HALF]], axis=0)
    stats = _tc_stats(out0, acc)
    if l < 3:
      out0, xw = _tc_layer(out0, acc, stats, gamma2, beta2, wrs[l + 1])

  x3, gate, m = _tc_gate(out0, acc, stats, gamma2, beta2, gate_w1, gb1,
                         gw2p, gb2p, batch3)
  lg = _tc_pool(x3, gate, m, batch3, gfp, h1, h2, bv)
  return lg[:, 0], lg[:, 1]


# speedup vs baseline: 10.2129x; 1.1597x over previous
import functools

import jax
import jax.numpy as jnp
from jax import lax
from jax.experimental import pallas as pl
from jax.experimental.pallas import tpu as pltpu
from jax.experimental.pallas import tpu_sc as plsc

_f32 = jnp.float32
_i32 = jnp.int32

N = 50000
E = 800000
R = 8
HID = 64
G = 16
NINE = (R + 1) * HID

HALF = 25000
PT = 1568
HPAD = 16 * PT
TRASH = HALF
CT = HPAD * 8
CZT = CT // 16
CTRASH = HALF * 8

EPAD = 802816
EROWS = EPAD // 128
RPT = EROWS // 16
NCHUNK = RPT // 4
RPT32 = EROWS // 32
NCHUNK32 = RPT32 // 4

BN_ = 2000
NB = N // BN_


def _sc_mesh():
  return plsc.VectorSubcoreMesh(core_axis_name="c", subcore_axis_name="s",
                                num_cores=2, num_subcores=16)


_SC_PARAMS = pltpu.CompilerParams(use_tc_tiling_on_sc=False)


def _sc_counts_body(cidx_ref, cnt_out_ref, cnt_sh, idxv, onesv, zb):
  c = lax.axis_index("c")
  s = lax.axis_index("s")
  zero16 = jnp.zeros((16,), _f32)

  def zl(i, carry):
    zb[pl.ds(i * 16, 16)] = zero16
    return carry

  lax.fori_loop(0, CZT // 16, zl, 0)
  for t in range(8):
    onesv[pl.ds(t * 16, 16)] = jnp.ones((16,), _f32)
  pltpu.sync_copy(zb, cnt_sh.at[pl.ds(s * CZT, CZT)])
  plsc.subcore_barrier()

  def chunk(i, carry):
    r0 = s * RPT + i * 4
    pltpu.sync_copy(cidx_ref.at[c, pl.ds(r0, 4)], idxv)
    for j in range(4):
      pltpu.sync_copy(onesv, cnt_sh.at[idxv.at[j]], add=True)
    return carry

  lax.fori_loop(0, NCHUNK, chunk, 0)
  plsc.subcore_barrier()
  pltpu.sync_copy(cnt_sh.at[pl.ds(s * CZT, CZT)],
                  cnt_out_ref.at[pl.ds(c * CT + s * CZT, CZT)])


def _sc_counts(cidx2):
  return pl.kernel(
      _sc_counts_body,
      out_type=jax.ShapeDtypeStruct((2 * CT,), _f32),
      mesh=_sc_mesh(),
      scratch_types=[
          pltpu.MemorySpace.VMEM_SHARED((CT,), _f32),
          pltpu.MemorySpace.VMEM((4, 128), _i32),
          pltpu.MemorySpace.VMEM((128,), _f32),
          pltpu.MemorySpace.VMEM((CZT,), _f32),
      ],
      compiler_params=_SC_PARAMS,
      name="sc_counts",
  )(cidx2)


def _sc_scale_body(sgidx_ref, cnt_ref, scale_out_ref, idxv, cv, sv, sem):
  c = lax.axis_index("c")
  s = lax.axis_index("s")
  w = s * 2 + c

  def chunk(i, carry):
    r0 = w * RPT32 + i * 4
    pltpu.sync_copy(sgidx_ref.at[pl.ds(r0, 4)], idxv)
    cps = [pltpu.async_copy(cnt_ref.at[idxv.at[j]], cv.at[j], sem)
           for j in range(4)]
    for cp in cps:
      cp.wait()
    for j in range(4):
      for t in range(8):
        sl = pl.ds(t * 16, 16)
        sv[j, sl] = 1.0 / jnp.maximum(cv[j, sl], 1.0)
    pltpu.sync_copy(sv, scale_out_ref.at[pl.ds(r0, 4)])
    return carry

  lax.fori_loop(0, NCHUNK32, chunk, 0)


def _sc_scale(sgidx2, cnt):
  return pl.kernel(
      _sc_scale_body,
      out_type=jax.ShapeDtypeStruct((EROWS, 128), _f32),
      mesh=_sc_mesh(),
      scratch_types=[
          pltpu.MemorySpace.VMEM((4, 128), _i32),
          pltpu.MemorySpace.VMEM((4, 128), _f32),
          pltpu.MemorySpace.VMEM((4, 128), _f32),
          pltpu.SemaphoreType.DMA,
      ],
      compiler_params=_SC_PARAMS,
      name="sc_scale",
  )(sgidx2, cnt)


def _sc_msg_body(xw_ref, gidx_ref, scl_ref, ldst_ref, acc_out_ref,
                 acc_sh, idxg, idxs, sclv, rows, semi, semg, sems):
  c = lax.axis_index("c")
  s = lax.axis_index("s")
  base = s * PT
  zero16 = jnp.zeros((16,), _f32)

  def zl(i, carry):
    for t in range(4):
      rows[i, pl.ds(t * 16, 16)] = zero16
    return carry

  lax.fori_loop(0, 256, zl, 0)
  for k in range(6):
    pltpu.sync_copy(rows, acc_sh.at[pl.ds(base + k * 256, 256)])
  pltpu.sync_copy(rows.at[pl.ds(0, PT - 1536)],
                  acc_sh.at[pl.ds(base + 1536, PT - 1536)])
  plsc.subcore_barrier()

  def scale_half(koff, half0):
    def grp(gi, gcarry):
      svec = sclv[pl.ds(koff * 128 + gi * 16, 16)]
      for j in range(16):
        sc = jnp.take_along_axis(svec, jnp.full((16,), j, _i32), axis=0)
        e = half0 + gi * 16 + j
        for t in range(4):
          sl = pl.ds(t * 16, 16)
          rows[e, sl] = rows[e, sl] * sc
      return gcarry

    lax.fori_loop(0, 8, grp, 0)

  def gather(k, half0):
    return pltpu.async_copy(xw_ref.at[idxg.at[k]],
                            rows.at[pl.ds(half0, 128)], semg)

  def scatter(k, half0):
    return pltpu.make_async_copy(rows.at[pl.ds(half0, 128)],
                                 acc_sh.at[idxs.at[k]], sems)

  def chunk(i, carry):
    r0 = s * RPT + i * 4
    ci = [pltpu.make_async_copy(gidx_ref.at[pl.ds(r0, 4)], idxg, semi),
          pltpu.make_async_copy(scl_ref.at[pl.ds(r0 * 128, 512)], sclv, semi),
          pltpu.make_async_copy(ldst_ref.at[c, pl.ds(r0, 4)], idxs, semi)]
    for cp in ci:
      cp.start()
    for cp in ci:
      cp.wait()
    g0 = gather(0, 0)
    g1 = gather(1, 128)
    g0.wait()
    scale_half(0, 0)
    s0 = scatter(0, 0)
    s0.start(add=True)
    g1.wait()
    scale_half(1, 128)
    s1 = scatter(1, 128)
    s1.start(add=True)
    s0.wait()
    g2 = gather(2, 0)
    g2.wait()
    scale_half(2, 0)
    s2 = scatter(2, 0)
    s2.start(add=True)
    s1.wait()
    g3 = gather(3, 128)
    g3.wait()
    scale_half(3, 128)
    s3 = scatter(3, 128)
    s3.start(add=True)
    s2.wait()
    s3.wait()
    return carry

  lax.fori_loop(0, NCHUNK, chunk, 0)
  plsc.subcore_barrier()
  pltpu.sync_copy(acc_sh.at[pl.ds(base, PT)],
                  acc_out_ref.at[pl.ds(c * HPAD + base, PT)])


def _sc_msg(xw_flat, gidx2, scale1, ldst2):
  return pl.kernel(
      _sc_msg_body,
      out_type=jax.ShapeDtypeStruct((2 * HPAD, HID), _f32),
      mesh=_sc_mesh(),
      scratch_types=[
          pltpu.MemorySpace.VMEM_SHARED((HPAD, HID), _f32),
          pltpu.MemorySpace.VMEM((4, 128), _i32),
          pltpu.MemorySpace.VMEM((4, 128), _i32),
          pltpu.MemorySpace.VMEM((512,), _f32),
          pltpu.MemorySpace.VMEM((256, HID), _f32),
          pltpu.SemaphoreType.DMA,
          pltpu.SemaphoreType.DMA,
          pltpu.SemaphoreType.DMA,
      ],
      compiler_params=_SC_PARAMS,
      name="sc_msg",
  )(xw_flat, gidx2, scale1, ldst2)


def _tc_layer1(xt3, xs3, bow, hot, topo, wa, wb, wcb, wch, wct):
  def body(xt_ref, xs_ref, bow_ref, hot_ref, topo_ref, wa_ref, wb_ref,
           wcb_ref, wch_ref, wct_ref, out0_ref, xw_ref):
    xt = xt_ref[0, 0, :]
    xs = xs_ref[0, 0, :]
    oht = (xt[:, None] == lax.broadcasted_iota(_i32, (1, 16), 1)).astype(_f32)
    ohs = (xs[:, None] == lax.broadcasted_iota(_i32, (1, 33), 1)).astype(_f32)
    res = (jnp.dot(oht, wa_ref[...], preferred_element_type=_f32)
           + jnp.dot(ohs, wb_ref[...], preferred_element_type=_f32)
           + jnp.dot(bow_ref[...], wcb_ref[...], preferred_element_type=_f32)
           + jnp.dot(hot_ref[...], wch_ref[...], preferred_element_type=_f32)
           + jnp.dot(topo_ref[...], wct_ref[...], preferred_element_type=_f32))
    out0_ref[...] = res[:, :HID]
    for r in range(R):
      xw_ref[r] = res[:, HID * (r + 1):HID * (r + 2)]

  return pl.pallas_call(
      body,
      grid=(NB,),
      in_specs=[
          pl.BlockSpec((1, 1, BN_), lambda i: (i, 0, 0)),
          pl.BlockSpec((1, 1, BN_), lambda i: (i, 0, 0)),
          pl.BlockSpec((BN_, 64), lambda i: (i, 0)),
          pl.BlockSpec((BN_, 10), lambda i: (i, 0)),
          pl.BlockSpec((BN_, 3), lambda i: (i, 0)),
          pl.BlockSpec((16, NINE), lambda i: (0, 0)),
          pl.BlockSpec((33, NINE), lambda i: (0, 0)),
          pl.BlockSpec((64, NINE), lambda i: (0, 0)),
          pl.BlockSpec((10, NINE), lambda i: (0, 0)),
          pl.BlockSpec((3, NINE), lambda i: (0, 0)),
      ],
      out_specs=[
          pl.BlockSpec((BN_, HID), lambda i: (i, 0)),
          pl.BlockSpec((R, BN_, HID), lambda i: (0, i, 0)),
      ],
      out_shape=[
          jax.ShapeDtypeStruct((N, HID), _f32),
          jax.ShapeDtypeStruct((R, N, HID), _f32),
      ],
      name="tc_layer1",
  )(xt3, xs3, bow, hot, topo, wa, wb, wcb, wch, wct)


def _norm_block(o_ref, a_ref, st_ref, g_ref, b_ref):
  h = o_ref[...] + a_ref[...]
  mu = st_ref[0:1, :] * (1.0 / N)
  msq = st_ref[1:2, :] * (1.0 / N)
  var = msq - mu * mu
  xn = g_ref[...] * (h - mu) * lax.rsqrt(var + 1e-5) + b_ref[...]
  return jnp.maximum(xn, 0.0)


def _tc_layer(out0p, accp, stats, gamma2, beta2, wr):
  def body(o_ref, a_ref, st_ref, g_ref, b_ref, wr_ref, out0_ref, xw_ref):
    x = _norm_block(o_ref, a_ref, st_ref, g_ref, b_ref)
    res = jnp.dot(x, wr_ref[...], preferred_element_type=_f32)
    out0_ref[...] = res[:, :HID]
    for r in range(R):
      xw_ref[r] = res[:, HID * (r + 1):HID * (r + 2)]

  return pl.pallas_call(
      body,
      grid=(NB,),
      in_specs=[
          pl.BlockSpec((BN_, HID), lambda i: (i, 0)),
          pl.BlockSpec((BN_, HID), lambda i: (i, 0)),
          pl.BlockSpec((8, HID), lambda i: (0, 0)),
          pl.BlockSpec((1, HID), lambda i: (0, 0)),
          pl.BlockSpec((1, HID), lambda i: (0, 0)),
          pl.BlockSpec((HID, NINE), lambda i: (0, 0)),
      ],
      out_specs=[
          pl.BlockSpec((BN_, HID), lambda i: (i, 0)),
          pl.BlockSpec((R, BN_, HID), lambda i: (0, i, 0)),
      ],
      out_shape=[
          jax.ShapeDtypeStruct((N, HID), _f32),
          jax.ShapeDtypeStruct((R, N, HID), _f32),
      ],
      name="tc_layer",
  )(out0p, accp, stats, gamma2, beta2, wr)


def _tc_stats(out0p, accp):
  def body(o_ref, a_ref, st_ref):
    i = pl.program_id(0)
    h = o_ref[...] + a_ref[...]

    @pl.when(i == 0)
    def _():
      st_ref[...] = jnp.zeros((8, HID), _f32)

    st_ref[0:1, :] = st_ref[0:1, :] + jnp.sum(h, axis=0, keepdims=True)
    st_ref[1:2, :] = st_ref[1:2, :] + jnp.sum(h * h, axis=0, keepdims=True)

  return pl.pallas_call(
      body,
      grid=(NB,),
      in_specs=[
          pl.BlockSpec((BN_, HID), lambda i: (i, 0)),
          pl.BlockSpec((BN_, HID), lambda i: (i, 0)),
      ],
      out_specs=pl.BlockSpec((8, HID), lambda i: (0, 0)),
      out_shape=jax.ShapeDtypeStruct((8, HID), _f32),
      name="tc_stats",
  )(out0p, accp)


def _tc_gate(out0p, accp, stats, gamma2, beta2, gw1, gb1, gw2p, gb2p, batch3):
  def body(o_ref, a_ref, st_ref, g_ref, b_ref, w1_ref, b1_ref, w2_ref,
           b2_ref, bt_ref, x3_ref, gate_ref, m_ref):
    i = pl.program_id(0)
    x = _norm_block(o_ref, a_ref, st_ref, g_ref, b_ref)
    x3_ref[...] = x
    g1 = jnp.maximum(
        jnp.dot(x, w1_ref[...], preferred_element_type=_f32) + b1_ref[...],
        0.0)
    gt = jnp.dot(g1, w2_ref[...], preferred_element_type=_f32) + b2_ref[...]
    gate_ref[...] = gt
    bt = bt_ref[0, 0, :]
    oh = bt[:, None] == lax.broadcasted_iota(_i32, (1, 128), 1)
    mp = jnp.max(jnp.where(oh, gt[:, 0:1], -1e30), axis=0, keepdims=True)

    @pl.when(i == 0)
    def _():
      m_ref[...] = jnp.full((8, 128), -1e30, _f32)

    m_ref[...] = jnp.maximum(m_ref[...], jnp.broadcast_to(mp, (8, 128)))

  return pl.pallas_call(
      body,
      grid=(NB,),
      in_specs=[
          pl.BlockSpec((BN_, HID), lambda i: (i, 0)),
          pl.BlockSpec((BN_, HID), lambda i: (i, 0)),
          pl.BlockSpec((8, HID), lambda i: (0, 0)),
          pl.BlockSpec((1, HID), lambda i: (0, 0)),
          pl.BlockSpec((1, HID), lambda i: (0, 0)),
          pl.BlockSpec((HID, 64), lambda i: (0, 0)),
          pl.BlockSpec((1, 64), lambda i: (0, 0)),
          pl.BlockSpec((64, 128), lambda i: (0, 0)),
          pl.BlockSpec((1, 128), lambda i: (0, 0)),
          pl.BlockSpec((1, 1, BN_), lambda i: (i, 0, 0)),
      ],
      out_specs=[
          pl.BlockSpec((BN_, HID), lambda i: (i, 0)),
          pl.BlockSpec((BN_, 128), lambda i: (i, 0)),
          pl.BlockSpec((8, 128), lambda i: (0, 0)),
      ],
      out_shape=[
          jax.ShapeDtypeStruct((N, HID), _f32),
          jax.ShapeDtypeStruct((N, 128), _f32),
          jax.ShapeDtypeStruct((8, 128), _f32),
      ],
      name="tc_gate",
  )(out0p, accp, stats, gamma2, beta2, gw1, gb1, gw2p, gb2p, batch3)


def _tc_pool(x3, gate, m, batch3, gfp, h1, h2, bv):
  def body(x_ref, gt_ref, m_ref, bt_ref, gf_ref, h1_ref, h2_ref, bv_ref,
           out_ref, zacc):
    i = pl.program_id(0)
    bt = bt_ref[0, 0, :]
    oh16 = (bt[:, None] == lax.broadcasted_iota(_i32, (1, G), 1)).astype(_f32)
    m16 = m_ref[0:1, 0:G]
    mb = jnp.sum(oh16 * m16, axis=1, keepdims=True)
    e = jnp.exp(gt_ref[:, 0:1] - mb)
    z = jnp.concatenate(
        [e * x_ref[...], jnp.broadcast_to(e, (BN_, HID))], axis=1)
    part = lax.dot_general(oh16, z, (((0,), (0,)), ((), ())),
                           preferred_element_type=_f32)

    @pl.when(i == 0)
    def _():
      zacc[...] = jnp.zeros((G, 128), _f32)

    zacc[...] = zacc[...] + part

    @pl.when(i == NB - 1)
    def _():
      u = zacc[...]
      sval = u[:, HID:HID + 1]
      gpool = u[:, 0:HID] / jnp.maximum(sval, 1e-30)
      lg = (jnp.dot(gpool, h1_ref[...], preferred_element_type=_f32)
            + jnp.dot(gf_ref[...], h2_ref[...], preferred_element_type=_f32)
            + bv_ref[...])
      out_ref[...] = lg

  return pl.pallas_call(
      body,
      grid=(NB,),
      in_specs=[
          pl.BlockSpec((BN_, HID), lambda i: (i, 0)),
          pl.BlockSpec((BN_, 128), lambda i: (i, 0)),
          pl.BlockSpec((8, 128), lambda i: (0, 0)),
          pl.BlockSpec((1, 1, BN_), lambda i: (i, 0, 0)),
          pl.BlockSpec((G, G), lambda i: (0, 0)),
          pl.BlockSpec((HID, 128), lambda i: (0, 0)),
          pl.BlockSpec((G, 128), lambda i: (0, 0)),
          pl.BlockSpec((1, 128), lambda i: (0, 0)),
      ],
      out_specs=pl.BlockSpec((G, 128), lambda i: (0, 0)),
      out_shape=jax.ShapeDtypeStruct((G, 128), _f32),
      scratch_shapes=[pltpu.MemorySpace.VMEM((G, 128), _f32)],
      name="tc_pool",
  )(x3, gate, m, batch3, gfp, h1, h2, bv)


def _wr_cat(comp, basis, root):
  w = jnp.einsum('rb,bio->rio', comp, basis)
  d = root.shape[0]
  return jnp.concatenate([root[None], w], axis=0).transpose(1, 0, 2).reshape(
      d, NINE)


def _pad_edges(a, fill):
  pad = EPAD - E
  return jnp.concatenate(
      [a, jnp.full((pad,), fill, a.dtype)]).reshape(EROWS, 128)


def kernel(x_type, x_sub, x_bow, x_hot, x_topo, edge_index, edge_type, batch,
           g_feats, emb_type, emb_sub, comp1, basis1, root1, bias1, comp2,
           basis2, root2, bias2, comp3, basis3, root3, bias3, bn_gamma,
           bn_beta, gate_w1, gate_b1, gate_w2, gate_b2, head_pt_w, head_pt_b,
           head_ci_w, head_ci_b):
  wr1 = _wr_cat(comp1, basis1, root1)
  wr2 = _wr_cat(comp2, basis2, root2)
  wr3 = _wr_cat(comp3, basis3, root3)
  wa = emb_type @ wr1[:16]
  wb = emb_sub @ wr1[16:24]
  wcb = wr1[24:88]
  wch = wr1[88:98]
  wct = wr1[98:101]
  gamma2 = bn_gamma.reshape(1, HID)
  beta2 = bn_beta.reshape(1, HID)
  gb1 = gate_b1.reshape(1, 64)
  gw2p = jnp.zeros((64, 128), _f32).at[:, 0].set(gate_w2[:, 0])
  gb2p = jnp.zeros((1, 128), _f32).at[0, 0].set(gate_b2[0])
  h1 = jnp.zeros((HID, 128), _f32)
  h1 = h1.at[:, 0].set(head_pt_w[:HID, 0]).at[:, 1].set(head_ci_w[:HID, 0])
  h2 = jnp.zeros((G, 128), _f32)
  h2 = h2.at[:12, 0].set(head_pt_w[HID:, 0]).at[:12, 1].set(head_ci_w[HID:, 0])
  gfp = jnp.zeros((G, G), _f32).at[:, :12].set(g_feats)
  bv = jnp.zeros((1, 128), _f32).at[0, 0].set(head_pt_b[0]).at[0, 1].set(
      head_ci_b[0])

  src = edge_index[0].astype(_i32)
  dst = edge_index[1].astype(_i32)
  et = edge_type.astype(_i32)
  in0 = dst < HALF
  gidx2 = _pad_edges(et * N + src, 0)
  ldst0 = jnp.where(in0, dst, TRASH)
  ldst1 = jnp.where(in0, TRASH, dst - HALF)
  ldst2 = jnp.stack([_pad_edges(ldst0, TRASH), _pad_edges(ldst1, TRASH)])
  cidx0 = jnp.where(in0, dst * 8 + et, CTRASH)
  cidx1 = jnp.where(in0, CTRASH, (dst - HALF) * 8 + et)
  cidx2 = jnp.stack([_pad_edges(cidx0, CTRASH), _pad_edges(cidx1, CTRASH)])
  cof = (~in0).astype(_i32)
  sgidx = cof * CT + (dst - cof * HALF) * 8 + et
  sgidx2 = _pad_edges(sgidx, 0)
  xt3 = x_type.astype(_i32).reshape(NB, 1, BN_)
  xs3 = x_sub.astype(_i32).reshape(NB, 1, BN_)
  batch3 = batch.astype(_i32).reshape(NB, 1, BN_)

  cnt = _sc_counts(cidx2)
  scale1 = _sc_scale(sgidx2, cnt).reshape(EPAD)

  out0, xw = _tc_layer1(xt3, xs3, x_bow, x_hot, x_topo,
                        wa, wb, wcb, wch, wct)
  wrs = (None, None, wr2, wr3)
  acc = None
  stats = None
  for l in (1, 2, 3):
    accp = _sc_msg(xw.reshape(R * N, HID), gidx2, scale1, ldst2)
    acc = jnp.concatenate([accp[:HALF], accp[HPAD:HPAD + HALF]], axis=0)
    stats = _tc_stats(out0, acc)
    if l < 3:
      out0, xw = _tc_layer(out0, acc, stats, gamma2, beta2, wrs[l + 1])

  x3, gate, m = _tc_gate(out0, acc, stats, gamma2, beta2, gate_w1, gb1,
                         gw2p, gb2p, batch3)
  lg = _tc_pool(x3, gate, m, batch3, gfp, h1, h2, bv)
  return lg[:, 0], lg[:, 1]
